# baseline (speedup 1.0000x reference)
import functools

import jax
import jax.numpy as jnp
import numpy as np
from jax import lax
from jax.experimental import pallas as pl
from jax.experimental.pallas import tpu as pltpu
from jax.experimental.pallas import tpu_sc as plsc

N = 10000
E = 320000
MUL = 128
RBF = 8
INV_SQRT = float(1.0 / np.sqrt(MUL))

NC, NS = 2, 16
NW = NC * NS
GW = 128
NP = 4
EH = E // NP
N_ACC = N + 8
ROWS_PS = 624
TAIL_BASE = ROWS_PS * NS
TAIL_ROWS = N - TAIL_BASE
ZROWS = 48
NSTEP_H = EH // GW

_mesh = plsc.VectorSubcoreMesh(core_axis_name="c", subcore_axis_name="s")


def _make_gather(half):
    step0 = half * NSTEP_H

    @functools.partial(
        pl.kernel,
        mesh=_mesh,
        out_type=jax.ShapeDtypeStruct((EH, MUL), jnp.float32),
    )
    def _gather(x_hbm, ei_hbm, out_hbm):
        def body(i_vmem, o_vmem):
            pltpu.sync_copy(x_hbm.at[i_vmem.at[0]], o_vmem)

        pltpu.emit_pipeline(
            body,
            grid=(NSTEP_H,),
            in_specs=[pl.BlockSpec((1, GW), lambda i, _o=step0: (0, i + _o))],
            out_specs=[pl.BlockSpec((GW, MUL), lambda i: (i, 0))],
            core_axis_name=("c", "s"),
            dimension_semantics=(pltpu.PARALLEL,),
        )(ei_hbm, out_hbm)

    return _gather


_sc_gathers = [_make_gather(q) for q in range(NP)]


def _make_scatter(half):
    step0 = half * NSTEP_H

    @functools.partial(
        pl.kernel,
        mesh=_mesh,
        out_type=jax.ShapeDtypeStruct((NC, 4, N, MUL), jnp.float32),
        scratch_types=[
            pltpu.VMEM((ZROWS, MUL), jnp.float32),
            pltpu.VMEM_SHARED((N_ACC, MUL), jnp.float32),
        ],
    )
    def _scatter(m_hbm, ei_hbm, out_hbm, zbuf_v, acc_sh):
        cid = lax.axis_index("c")
        sid = lax.axis_index("s")

        @pl.loop(0, ZROWS)
        def _(r):
            for t in range(MUL // 16):
                zbuf_v[r, pl.ds(t * 16, 16)] = jnp.zeros((16,), jnp.float32)

        for comp in range(4):
            for i in range(ROWS_PS // ZROWS):
                pltpu.sync_copy(
                    zbuf_v, acc_sh.at[pl.ds(sid * ROWS_PS + i * ZROWS, ZROWS)]
                )

            @pl.when(sid == NS - 1)
            def _():
                pltpu.sync_copy(
                    zbuf_v.at[pl.ds(0, TAIL_ROWS)],
                    acc_sh.at[pl.ds(TAIL_BASE, TAIL_ROWS)],
                )

            plsc.subcore_barrier()

            def body(i_vmem, rows_vmem):
                pltpu.sync_copy(
                    rows_vmem.at[0], acc_sh.at[i_vmem.at[0]], add=True
                )

            pltpu.emit_pipeline(
                body,
                grid=(NSTEP_H,),
                in_specs=[
                    pl.BlockSpec((1, GW), lambda i, _o=step0: (1, i + _o)),
                    pl.BlockSpec((1, GW, MUL), lambda i, _c=comp: (_c, i, 0)),
                ],
                out_specs=[],
                core_axis_name=("c", "s"),
                dimension_semantics=(pltpu.PARALLEL,),
            )(ei_hbm, m_hbm)

            plsc.subcore_barrier()
            pltpu.sync_copy(
                acc_sh.at[pl.ds(sid * ROWS_PS, ROWS_PS)],
                out_hbm.at[cid, comp, pl.ds(sid * ROWS_PS, ROWS_PS)],
            )

            @pl.when(sid == NS - 1)
            def _():
                pltpu.sync_copy(
                    acc_sh.at[pl.ds(TAIL_BASE, TAIL_ROWS)],
                    out_hbm.at[cid, comp, pl.ds(TAIL_BASE, TAIL_ROWS)],
                )

            plsc.subcore_barrier()

    return _scatter


_sc_scatters = [_make_scatter(q) for q in range(NP)]


def _pre_body(nf_ref, w_ref, o_ref):
    o_ref[...] = (
        jnp.dot(nf_ref[...], w_ref[...], preferred_element_type=jnp.float32)
        * INV_SQRT
    )


_tc_pre = pl.pallas_call(
    _pre_body,
    out_shape=jax.ShapeDtypeStruct((N, MUL), jnp.float32),
)

BE = 3200
_DN_T = (((0,), (0,)), ((), ()))


def _msg_body(rbt_ref, spht_ref, ng_ref, w1_ref, w2_ref, w3_ref, m_ref):
    h = lax.dot_general(rbt_ref[...], w1_ref[...], _DN_T,
                        preferred_element_type=jnp.float32)
    h = h * jax.nn.sigmoid(h)
    h = jnp.dot(h, w2_ref[...], preferred_element_type=jnp.float32)
    h = h * jax.nn.sigmoid(h)
    w = jnp.dot(h, w3_ref[...], preferred_element_type=jnp.float32)
    ys = lax.dot_general(spht_ref[...], jnp.eye(4, dtype=jnp.float32), _DN_T,
                         preferred_element_type=jnp.float32)
    ng = ng_ref[...]
    t0 = w[:, :MUL] * ng
    t1 = w[:, MUL:] * ng
    m_ref[0] = t0 * ys[:, 0:1]
    m_ref[1] = t1 * ys[:, 1:2]
    m_ref[2] = t1 * ys[:, 2:3]
    m_ref[3] = t1 * ys[:, 3:4]


def _make_messages(half):
    ofs = half * (EH // BE)
    return pl.pallas_call(
        _msg_body,
        grid=(EH // BE,),
        in_specs=[
            pl.BlockSpec((RBF, BE), lambda i, _o=ofs: (0, i + _o)),
            pl.BlockSpec((4, BE), lambda i, _o=ofs: (0, i + _o)),
            pl.BlockSpec((BE, MUL), lambda i: (i, 0)),
            pl.BlockSpec((RBF, 64), lambda i: (0, 0)),
            pl.BlockSpec((64, 64), lambda i: (0, 0)),
            pl.BlockSpec((64, 2 * MUL), lambda i: (0, 0)),
        ],
        out_specs=pl.BlockSpec((4, BE, MUL), lambda i: (0, i, 0)),
        out_shape=jax.ShapeDtypeStruct((4, EH, MUL), jnp.float32),
    )


_tc_messages = [_make_messages(q) for q in range(NP)]

BN = 1000


def _post_body(p0_ref, p1_ref, p2_ref, p3_ref, ws_ref, wv_ref, o_ref):
    t = (p0_ref[0] + p0_ref[1] + p1_ref[0] + p1_ref[1]
         + p2_ref[0] + p2_ref[1] + p3_ref[0] + p3_ref[1])
    ws = ws_ref[...] * INV_SQRT
    wv = wv_ref[...] * INV_SQRT
    o_ref[0] = jnp.dot(t[0], ws, preferred_element_type=jnp.float32)
    for comp in range(3):
        o_ref[1 + comp] = jnp.dot(
            t[1 + comp], wv, preferred_element_type=jnp.float32
        )


_tc_post = pl.pallas_call(
    _post_body,
    grid=(N // BN,),
    in_specs=[
        pl.BlockSpec((NC, 4, BN, MUL), lambda i: (0, 0, i, 0)),
        pl.BlockSpec((NC, 4, BN, MUL), lambda i: (0, 0, i, 0)),
        pl.BlockSpec((NC, 4, BN, MUL), lambda i: (0, 0, i, 0)),
        pl.BlockSpec((NC, 4, BN, MUL), lambda i: (0, 0, i, 0)),
        pl.BlockSpec((MUL, MUL), lambda i: (0, 0)),
        pl.BlockSpec((MUL, MUL), lambda i: (0, 0)),
    ],
    out_specs=pl.BlockSpec((4, BN, MUL), lambda i: (0, i, 0)),
    out_shape=jax.ShapeDtypeStruct((4, N, MUL), jnp.float32),
)


def kernel(node_features, sph_harmonics, radial_basis, edge_index,
           W_pre, W1, W2, W3, W_post_s, W_post_v):
    rbt = radial_basis.T
    spht = sph_harmonics.T
    x = _tc_pre(node_features, W_pre)
    neighs = [g(x, edge_index) for g in _sc_gathers]
    parts = []
    for q in range(NP):
        m = _tc_messages[q](rbt, spht, neighs[q], W1, W2, W3)
        parts.append(_sc_scatters[q](m, edge_index))
    out4 = _tc_post(*parts, W_post_s, W_post_v)
    return jnp.transpose(out4, (1, 2, 0))


# speedup vs baseline: 1.0267x; 1.0267x over previous
import functools

import jax
import jax.numpy as jnp
import numpy as np
from jax import lax
from jax.experimental import pallas as pl
from jax.experimental.pallas import tpu as pltpu
from jax.experimental.pallas import tpu_sc as plsc

N = 10000
E = 320000
MUL = 128
RBF = 8
INV_SQRT = float(1.0 / np.sqrt(MUL))

NC, NS = 2, 16
NW = NC * NS
GW = 128
E0 = E // 4
E1 = E - E0
N_ACC = N + 8
ROWS_PS = 624
TAIL_BASE = ROWS_PS * NS
TAIL_ROWS = N - TAIL_BASE
ZROWS = 48
STEP1 = E0 // GW

_mesh = plsc.VectorSubcoreMesh(core_axis_name="c", subcore_axis_name="s")


def _make_gather(e_len, step0):
    nstep = e_len // GW

    @functools.partial(
        pl.kernel,
        mesh=_mesh,
        out_type=jax.ShapeDtypeStruct((e_len, MUL), jnp.float32),
    )
    def _gather(x_hbm, ei_hbm, out_hbm):
        def body(i_vmem, o_vmem):
            pltpu.sync_copy(x_hbm.at[i_vmem.at[0]], o_vmem)

        pltpu.emit_pipeline(
            body,
            grid=(nstep,),
            in_specs=[pl.BlockSpec((1, GW), lambda i, _o=step0: (0, i + _o))],
            out_specs=[pl.BlockSpec((GW, MUL), lambda i: (i, 0))],
            core_axis_name=("c", "s"),
            dimension_semantics=(pltpu.PARALLEL,),
        )(ei_hbm, out_hbm)

    return _gather


_sc_gather_0 = _make_gather(E0, 0)
_sc_gather_1 = _make_gather(E1, STEP1)


def _make_scatter(e_len, step0):
    nstep = e_len // GW

    @functools.partial(
        pl.kernel,
        mesh=_mesh,
        out_type=jax.ShapeDtypeStruct((NC, 4, N, MUL), jnp.float32),
        scratch_types=[
            pltpu.VMEM((ZROWS, MUL), jnp.float32),
            pltpu.VMEM_SHARED((N_ACC, MUL), jnp.float32),
        ],
    )
    def _scatter(m_hbm, ei_hbm, out_hbm, zbuf_v, acc_sh):
        cid = lax.axis_index("c")
        sid = lax.axis_index("s")

        @pl.loop(0, ZROWS)
        def _(r):
            for t in range(MUL // 16):
                zbuf_v[r, pl.ds(t * 16, 16)] = jnp.zeros((16,), jnp.float32)

        for comp in range(4):
            for i in range(ROWS_PS // ZROWS):
                pltpu.sync_copy(
                    zbuf_v, acc_sh.at[pl.ds(sid * ROWS_PS + i * ZROWS, ZROWS)]
                )

            @pl.when(sid == NS - 1)
            def _():
                pltpu.sync_copy(
                    zbuf_v.at[pl.ds(0, TAIL_ROWS)],
                    acc_sh.at[pl.ds(TAIL_BASE, TAIL_ROWS)],
                )

            plsc.subcore_barrier()

            def body(i_vmem, rows_vmem):
                pltpu.sync_copy(
                    rows_vmem.at[0], acc_sh.at[i_vmem.at[0]], add=True
                )

            pltpu.emit_pipeline(
                body,
                grid=(nstep,),
                in_specs=[
                    pl.BlockSpec((1, GW), lambda i, _o=step0: (1, i + _o)),
                    pl.BlockSpec((1, GW, MUL), lambda i, _c=comp: (_c, i, 0)),
                ],
                out_specs=[],
                core_axis_name=("c", "s"),
                dimension_semantics=(pltpu.PARALLEL,),
            )(ei_hbm, m_hbm)

            plsc.subcore_barrier()
            pltpu.sync_copy(
                acc_sh.at[pl.ds(sid * ROWS_PS, ROWS_PS)],
                out_hbm.at[cid, comp, pl.ds(sid * ROWS_PS, ROWS_PS)],
            )

            @pl.when(sid == NS - 1)
            def _():
                pltpu.sync_copy(
                    acc_sh.at[pl.ds(TAIL_BASE, TAIL_ROWS)],
                    out_hbm.at[cid, comp, pl.ds(TAIL_BASE, TAIL_ROWS)],
                )

            plsc.subcore_barrier()

    return _scatter


_sc_scatter_0 = _make_scatter(E0, 0)
_sc_scatter_1 = _make_scatter(E1, STEP1)


def _pre_body(nf_ref, w_ref, o_ref):
    o_ref[...] = (
        jnp.dot(nf_ref[...], w_ref[...], preferred_element_type=jnp.float32)
        * INV_SQRT
    )


_tc_pre = pl.pallas_call(
    _pre_body,
    out_shape=jax.ShapeDtypeStruct((N, MUL), jnp.float32),
)

BE = 3200
_DN_T = (((0,), (0,)), ((), ()))


def _msg_body(rbt_ref, spht_ref, ng_ref, w1_ref, w2_ref, w3_ref, m_ref):
    h = lax.dot_general(rbt_ref[...], w1_ref[...], _DN_T,
                        preferred_element_type=jnp.float32)
    h = h * jax.nn.sigmoid(h)
    h = jnp.dot(h, w2_ref[...], preferred_element_type=jnp.float32)
    h = h * jax.nn.sigmoid(h)
    w = jnp.dot(h, w3_ref[...], preferred_element_type=jnp.float32)
    ys = lax.dot_general(spht_ref[...], jnp.eye(4, dtype=jnp.float32), _DN_T,
                         preferred_element_type=jnp.float32)
    ng = ng_ref[...]
    t0 = w[:, :MUL] * ng
    t1 = w[:, MUL:] * ng
    m_ref[0] = t0 * ys[:, 0:1]
    m_ref[1] = t1 * ys[:, 1:2]
    m_ref[2] = t1 * ys[:, 2:3]
    m_ref[3] = t1 * ys[:, 3:4]


def _make_messages(e_len, ofs):
    return pl.pallas_call(
        _msg_body,
        grid=(e_len // BE,),
        in_specs=[
            pl.BlockSpec((RBF, BE), lambda i, _o=ofs: (0, i + _o)),
            pl.BlockSpec((4, BE), lambda i, _o=ofs: (0, i + _o)),
            pl.BlockSpec((BE, MUL), lambda i: (i, 0)),
            pl.BlockSpec((RBF, 64), lambda i: (0, 0)),
            pl.BlockSpec((64, 64), lambda i: (0, 0)),
            pl.BlockSpec((64, 2 * MUL), lambda i: (0, 0)),
        ],
        out_specs=pl.BlockSpec((4, BE, MUL), lambda i: (0, i, 0)),
        out_shape=jax.ShapeDtypeStruct((4, e_len, MUL), jnp.float32),
    )


_tc_messages_0 = _make_messages(E0, 0)
_tc_messages_1 = _make_messages(E1, E0 // BE)

BN = 2000


def _post_body(p0_ref, p1_ref, ws_ref, wv_ref, o_ref):
    t = p0_ref[0] + p0_ref[1] + p1_ref[0] + p1_ref[1]
    ws = ws_ref[...] * INV_SQRT
    wv = wv_ref[...] * INV_SQRT
    o_ref[0] = jnp.dot(t[0], ws, preferred_element_type=jnp.float32)
    for comp in range(3):
        o_ref[1 + comp] = jnp.dot(
            t[1 + comp], wv, preferred_element_type=jnp.float32
        )


_tc_post = pl.pallas_call(
    _post_body,
    grid=(N // BN,),
    in_specs=[
        pl.BlockSpec((NC, 4, BN, MUL), lambda i: (0, 0, i, 0)),
        pl.BlockSpec((NC, 4, BN, MUL), lambda i: (0, 0, i, 0)),
        pl.BlockSpec((MUL, MUL), lambda i: (0, 0)),
        pl.BlockSpec((MUL, MUL), lambda i: (0, 0)),
    ],
    out_specs=pl.BlockSpec((4, BN, MUL), lambda i: (0, i, 0)),
    out_shape=jax.ShapeDtypeStruct((4, N, MUL), jnp.float32),
)


def kernel(node_features, sph_harmonics, radial_basis, edge_index,
           W_pre, W1, W2, W3, W_post_s, W_post_v):
    rbt = radial_basis.T
    spht = sph_harmonics.T
    x = _tc_pre(node_features, W_pre)
    neigh0 = _sc_gather_0(x, edge_index)
    neigh1 = _sc_gather_1(x, edge_index)
    m0 = _tc_messages_0(rbt, spht, neigh0, W1, W2, W3)
    p0 = _sc_scatter_0(m0, edge_index)
    m1 = _tc_messages_1(rbt, spht, neigh1, W1, W2, W3)
    p1 = _sc_scatter_1(m1, edge_index)
    out4 = _tc_post(p0, p1, W_post_s, W_post_v)
    return jnp.transpose(out4, (1, 2, 0))


# speedup vs baseline: 1.0882x; 1.0599x over previous
import functools

import jax
import jax.numpy as jnp
import numpy as np
from jax import lax
from jax.experimental import pallas as pl
from jax.experimental.pallas import tpu as pltpu
from jax.experimental.pallas import tpu_sc as plsc

N = 10000
E = 320000
MUL = 128
RBF = 8
INV_SQRT = float(1.0 / np.sqrt(MUL))

NC, NS = 2, 16
NW = NC * NS
GW = 128
EH = E // 2
N_ACC = N + 8
ROWS_PS = 624
TAIL_BASE = ROWS_PS * NS
TAIL_ROWS = N - TAIL_BASE
ZROWS = 48
NSTEP = E // GW
NSTEP_H = EH // GW

_mesh = plsc.VectorSubcoreMesh(core_axis_name="c", subcore_axis_name="s")


def _make_gather(half):
    step0 = half * NSTEP_H

    @functools.partial(
        pl.kernel,
        mesh=_mesh,
        out_type=jax.ShapeDtypeStruct((EH, MUL), jnp.float32),
    )
    def _gather(x_hbm, ei_hbm, out_hbm):
        def body(i_vmem, o_vmem):
            pltpu.sync_copy(x_hbm.at[i_vmem.at[0]], o_vmem)

        pltpu.emit_pipeline(
            body,
            grid=(NSTEP_H,),
            in_specs=[pl.BlockSpec((1, GW), lambda i, _o=step0: (0, i + _o))],
            out_specs=[pl.BlockSpec((GW, MUL), lambda i: (i, 0))],
            core_axis_name=("c", "s"),
            dimension_semantics=(pltpu.PARALLEL,),
        )(ei_hbm, out_hbm)

    return _gather


_sc_gather_0 = _make_gather(0)
_sc_gather_1 = _make_gather(1)


def _make_scatter(half):
    step0 = half * NSTEP_H

    @functools.partial(
        pl.kernel,
        mesh=_mesh,
        out_type=jax.ShapeDtypeStruct((NC, 4, N, MUL), jnp.float32),
        scratch_types=[
            pltpu.VMEM((ZROWS, MUL), jnp.float32),
            pltpu.VMEM_SHARED((N_ACC, MUL), jnp.float32),
        ],
    )
    def _scatter(m_hbm, ei_hbm, out_hbm, zbuf_v, acc_sh):
        cid = lax.axis_index("c")
        sid = lax.axis_index("s")

        @pl.loop(0, ZROWS)
        def _(r):
            for t in range(MUL // 16):
                zbuf_v[r, pl.ds(t * 16, 16)] = jnp.zeros((16,), jnp.float32)

        for comp in range(4):
            for i in range(ROWS_PS // ZROWS):
                pltpu.sync_copy(
                    zbuf_v, acc_sh.at[pl.ds(sid * ROWS_PS + i * ZROWS, ZROWS)]
                )

            @pl.when(sid == NS - 1)
            def _():
                pltpu.sync_copy(
                    zbuf_v.at[pl.ds(0, TAIL_ROWS)],
                    acc_sh.at[pl.ds(TAIL_BASE, TAIL_ROWS)],
                )

            plsc.subcore_barrier()

            def body(i_vmem, rows_vmem):
                pltpu.sync_copy(
                    rows_vmem.at[0], acc_sh.at[i_vmem.at[0]], add=True
                )

            pltpu.emit_pipeline(
                body,
                grid=(NSTEP_H,),
                in_specs=[
                    pl.BlockSpec((1, GW), lambda i, _o=step0: (1, i + _o)),
                    pl.BlockSpec((1, GW, MUL), lambda i, _c=comp: (_c, i, 0)),
                ],
                out_specs=[],
                core_axis_name=("c", "s"),
                dimension_semantics=(pltpu.PARALLEL,),
            )(ei_hbm, m_hbm)

            plsc.subcore_barrier()
            pltpu.sync_copy(
                acc_sh.at[pl.ds(sid * ROWS_PS, ROWS_PS)],
                out_hbm.at[cid, comp, pl.ds(sid * ROWS_PS, ROWS_PS)],
            )

            @pl.when(sid == NS - 1)
            def _():
                pltpu.sync_copy(
                    acc_sh.at[pl.ds(TAIL_BASE, TAIL_ROWS)],
                    out_hbm.at[cid, comp, pl.ds(TAIL_BASE, TAIL_ROWS)],
                )

            plsc.subcore_barrier()

    return _scatter


_sc_scatter_0 = _make_scatter(0)
_sc_scatter_1 = _make_scatter(1)


def _pre_body(nf_ref, w_ref, o_ref):
    o_ref[...] = (
        jnp.dot(nf_ref[...], w_ref[...], preferred_element_type=jnp.float32)
        * INV_SQRT
    )


_tc_pre = pl.pallas_call(
    _pre_body,
    out_shape=jax.ShapeDtypeStruct((N, MUL), jnp.float32),
)

BE = 3200
_DN_T = (((0,), (0,)), ((), ()))


def _msg_body(rbt_ref, spht_ref, ng_ref, w1_ref, w2_ref, w3_ref, m_ref):
    h = lax.dot_general(rbt_ref[...], w1_ref[...], _DN_T,
                        preferred_element_type=jnp.float32)
    h = h * jax.nn.sigmoid(h)
    h = jnp.dot(h, w2_ref[...], preferred_element_type=jnp.float32)
    h = h * jax.nn.sigmoid(h)
    w = jnp.dot(h, w3_ref[...], preferred_element_type=jnp.float32)
    ys = lax.dot_general(spht_ref[...], jnp.eye(4, dtype=jnp.float32), _DN_T,
                         preferred_element_type=jnp.float32)
    ng = ng_ref[...]
    t0 = w[:, :MUL] * ng
    t1 = w[:, MUL:] * ng
    m_ref[0] = t0 * ys[:, 0:1]
    m_ref[1] = t1 * ys[:, 1:2]
    m_ref[2] = t1 * ys[:, 2:3]
    m_ref[3] = t1 * ys[:, 3:4]


def _make_messages(half):
    ofs = half * (EH // BE)
    return pl.pallas_call(
        _msg_body,
        grid=(EH // BE,),
        in_specs=[
            pl.BlockSpec((RBF, BE), lambda i, _o=ofs: (0, i + _o)),
            pl.BlockSpec((4, BE), lambda i, _o=ofs: (0, i + _o)),
            pl.BlockSpec((BE, MUL), lambda i: (i, 0)),
            pl.BlockSpec((RBF, 64), lambda i: (0, 0)),
            pl.BlockSpec((64, 64), lambda i: (0, 0)),
            pl.BlockSpec((64, 2 * MUL), lambda i: (0, 0)),
        ],
        out_specs=pl.BlockSpec((4, BE, MUL), lambda i: (0, i, 0)),
        out_shape=jax.ShapeDtypeStruct((4, EH, MUL), jnp.float32),
    )


_tc_messages_0 = _make_messages(0)
_tc_messages_1 = _make_messages(1)

BN = 2000


def _post_body(p0_ref, p1_ref, ws_ref, wv_ref, o_ref):
    t = p0_ref[0] + p0_ref[1] + p1_ref[0] + p1_ref[1]
    ws = ws_ref[...] * INV_SQRT
    wv = wv_ref[...] * INV_SQRT
    o_ref[0] = jnp.dot(t[0], ws, preferred_element_type=jnp.float32)
    for comp in range(3):
        o_ref[1 + comp] = jnp.dot(
            t[1 + comp], wv, preferred_element_type=jnp.float32
        )


_tc_post = pl.pallas_call(
    _post_body,
    grid=(N // BN,),
    in_specs=[
        pl.BlockSpec((NC, 4, BN, MUL), lambda i: (0, 0, i, 0)),
        pl.BlockSpec((NC, 4, BN, MUL), lambda i: (0, 0, i, 0)),
        pl.BlockSpec((MUL, MUL), lambda i: (0, 0)),
        pl.BlockSpec((MUL, MUL), lambda i: (0, 0)),
    ],
    out_specs=pl.BlockSpec((4, BN, MUL), lambda i: (0, i, 0)),
    out_shape=jax.ShapeDtypeStruct((4, N, MUL), jnp.float32),
)


def kernel(node_features, sph_harmonics, radial_basis, edge_index,
           W_pre, W1, W2, W3, W_post_s, W_post_v):
    rbt = radial_basis.T
    spht = sph_harmonics.T
    x = _tc_pre(node_features, W_pre)
    neigh0 = _sc_gather_0(x, edge_index)
    neigh1 = _sc_gather_1(x, edge_index)
    m0 = _tc_messages_0(rbt, spht, neigh0, W1, W2, W3)
    p0 = _sc_scatter_0(m0, edge_index)
    m1 = _tc_messages_1(rbt, spht, neigh1, W1, W2, W3)
    p1 = _sc_scatter_1(m1, edge_index)
    out4 = _tc_post(p0, p1, W_post_s, W_post_v)
    return jnp.transpose(out4, (1, 2, 0))
